# baseline (speedup 1.0000x reference)
import functools

import jax
import jax.numpy as jnp
from jax.experimental import pallas as pl
from jax.experimental.pallas import tpu as pltpu

_MEMORY_DIM = 128
_MEMORY_SIZE = 64
_TEMPERATURE = 0.07
_EPS = 1e-12


def _ema_block_kernel(x_ref, mb_ref, o_ref):
    blk_b, blk_s, dim = x_ref.shape
    x = x_ref[...].reshape(blk_b * blk_s, dim)
    mb = mb_ref[...]

    mb_n = jnp.sqrt(jnp.sum(mb * mb, axis=1, keepdims=True))
    mb = mb / jnp.maximum(mb_n, _EPS)

    ones_bb = jnp.ones((_MEMORY_DIM, _MEMORY_DIM), dtype=jnp.float32)
    n2 = jnp.dot(x * x, ones_bb, preferred_element_type=jnp.float32)
    raw = jnp.dot(x, mb.T * (1.0 / _TEMPERATURE), preferred_element_type=jnp.float32)
    inv_n = jax.lax.rsqrt(jnp.maximum(n2, _EPS * _EPS))

    e = jnp.exp(raw * inv_n[:, :_MEMORY_SIZE])
    z = jnp.sum(e, axis=1, keepdims=True)
    a = e / z

    r = jnp.dot(a, mb, preferred_element_type=jnp.float32)
    o_ref[...] = (x * inv_n + r).reshape(blk_b, blk_s, dim)


@functools.partial(jax.jit, static_argnames=("block_batch",))
def _ema_forward(features, memory_bank, block_batch=4):
    batch, seq, dim = features.shape
    grid = (batch // block_batch,)

    return pl.pallas_call(
        _ema_block_kernel,
        grid=grid,
        in_specs=[
            pl.BlockSpec((block_batch, seq, dim), lambda i: (i, 0, 0)),
            pl.BlockSpec((_MEMORY_SIZE, dim), lambda i: (0, 0)),
        ],
        out_specs=pl.BlockSpec((block_batch, seq, dim), lambda i: (i, 0, 0)),
        out_shape=jax.ShapeDtypeStruct((batch, seq, dim), features.dtype),
        compiler_params=pltpu.CompilerParams(
            dimension_semantics=("parallel",),
        ),
    )(features, memory_bank)


def kernel(features, memory_bank):
    return _ema_forward(features, memory_bank)


# speedup vs baseline: 1.0072x; 1.0072x over previous
import functools

import jax
import jax.numpy as jnp
from jax.experimental import pallas as pl
from jax.experimental.pallas import tpu as pltpu

_MEMORY_DIM = 128
_MEMORY_SIZE = 64
_TEMPERATURE = 0.07
_EPS = 1e-12


def _ema_block_kernel(x_ref, mb_ref, o_ref):
    blk_b, blk_s, dim = x_ref.shape
    x = x_ref[...].reshape(blk_b * blk_s, dim)
    mb = mb_ref[...]

    mb_n = jnp.sqrt(jnp.sum(mb * mb, axis=1, keepdims=True))
    mb = mb / jnp.maximum(mb_n, _EPS)

    ones_bb = jnp.ones((_MEMORY_DIM, _MEMORY_DIM), dtype=jnp.float32)
    n2 = jnp.dot(x * x, ones_bb, preferred_element_type=jnp.float32)
    inv_n = jax.lax.rsqrt(jnp.maximum(n2, _EPS * _EPS))
    xn = x * inv_n

    log2e = 1.4426950408889634
    s2 = jnp.dot(xn, mb.T * (log2e / _TEMPERATURE), preferred_element_type=jnp.float32)
    e = jnp.exp2(s2)
    z = jnp.sum(e, axis=1, keepdims=True)
    a = e / z

    r = jnp.dot(a, mb, preferred_element_type=jnp.float32)
    o_ref[...] = (xn + r).reshape(blk_b, blk_s, dim)


@functools.partial(jax.jit, static_argnames=("block_batch",))
def _ema_forward(features, memory_bank, block_batch=4):
    batch, seq, dim = features.shape
    grid = (batch // block_batch,)

    return pl.pallas_call(
        _ema_block_kernel,
        grid=grid,
        in_specs=[
            pl.BlockSpec((block_batch, seq, dim), lambda i: (i, 0, 0)),
            pl.BlockSpec((_MEMORY_SIZE, dim), lambda i: (0, 0)),
        ],
        out_specs=pl.BlockSpec((block_batch, seq, dim), lambda i: (i, 0, 0)),
        out_shape=jax.ShapeDtypeStruct((batch, seq, dim), features.dtype),
        compiler_params=pltpu.CompilerParams(
            dimension_semantics=("parallel",),
        ),
    )(features, memory_bank)


def kernel(features, memory_bank):
    return _ema_forward(features, memory_bank)
